# Initial kernel scaffold; baseline (speedup 1.0000x reference)
import functools

import jax
import jax.numpy as jnp
from jax import lax
from jax.experimental import pallas as pl
from jax.experimental.pallas import tpu as pltpu
from jax.experimental.pallas import tpu_sc as plsc

N_P = 10000
N_L = 10000
E = 160000
D = 256
HID = 256
H = 4
RBF_DIM = 16
SIGMA = 4.0

NC = 2
NS = 16
NW = NC * NS
CHUNK = 128
TILE_E = 5120
NCHUNK = TILE_E // CHUNK
E_PAD = NW * TILE_E
BE = 512
GRID_E = E_PAD // BE
NL_PAD = 10048
HALF = N_L // 2
AGG_PAD = 5120
GEO = 32
PW = 16


def _widx():
  return lax.axis_index("c") * NS + lax.axis_index("s")


def _sc_gather(tabp, tabl, posp, posl, pidx3, lidx3,
               hp_out, hl_out, rp_out, rl_out,
               idxp, idxl, bufp, bufl, bufrp, bufrl,
               sem0, sem1, sem2, sem3):
  w = _widx()
  pltpu.sync_copy(pidx3.at[w], idxp)
  pltpu.sync_copy(lidx3.at[w], idxl)
  row0 = w * TILE_E

  def body(j, carry):
    r = row0 + j * CHUNK
    cp0 = pltpu.async_copy(tabp.at[idxp.at[j]], bufp, sem0)
    cp1 = pltpu.async_copy(tabl.at[idxl.at[j]], bufl, sem1)
    cp2 = pltpu.async_copy(posp.at[idxp.at[j]], bufrp, sem2)
    cp3 = pltpu.async_copy(posl.at[idxl.at[j]], bufrl, sem3)
    cp0.wait()
    pltpu.sync_copy(bufp, hp_out.at[pl.ds(r, CHUNK)])
    cp1.wait()
    pltpu.sync_copy(bufl, hl_out.at[pl.ds(r, CHUNK)])
    cp2.wait()
    pltpu.sync_copy(bufrp, rp_out.at[pl.ds(r, CHUNK)])
    cp3.wait()
    pltpu.sync_copy(bufrl, rl_out.at[pl.ds(r, CHUNK)])
    return carry

  lax.fori_loop(0, NCHUNK, body, 0)


def _geometry(rp, rl):
  diff = rl - rp
  d2 = jnp.sum(diff * diff, axis=1, keepdims=True)
  dist = jnp.sqrt(d2)
  dirs = diff * (1.0 / (dist + 1e-8))
  centers = lax.broadcasted_iota(jnp.float32, (BE, RBF_DIM), 1) * (
      8.0 / (RBF_DIM - 1))
  gamma_rbf = 1.0 / (2.0 * (8.0 / RBF_DIM) ** 2)
  rbf = jnp.exp(-gamma_rbf * (dist - centers) ** 2)
  geo = jnp.concatenate(
      [dist, dirs[:, :3], rbf, dirs[:, 0:1], dirs[:, 1:2],
       jnp.zeros((BE, GEO - 22), jnp.float32)], axis=1)
  return geo, d2


def _tc_att(hp_ref, hl_ref, rp_ref, rl_ref,
            w1p_ref, w1l_ref, w1g_ref, b1_ref, w2_ref, b2_ref,
            elog_ref, numer_ref):
  hp = hp_ref[...]
  hl = hl_ref[...]
  geo, d2 = _geometry(rp_ref[...], rl_ref[...])
  decay = jnp.exp(d2 * (-1.0 / (2.0 * SIGMA * SIGMA)))
  cols = []
  for h in range(H):
    acc = (jnp.dot(hp, w1p_ref[h], preferred_element_type=jnp.float32)
           + jnp.dot(hl, w1l_ref[h], preferred_element_type=jnp.float32)
           + jnp.dot(geo, w1g_ref[h], preferred_element_type=jnp.float32)
           + b1_ref[h])
    z = acc * jax.nn.sigmoid(acc)
    lg = jnp.sum(z * w2_ref[h], axis=1, keepdims=True) + b2_ref[h]
    cols.append(jnp.exp(lg))
  elog = jnp.concatenate(
      cols + [jnp.zeros((BE, 16 - H), jnp.float32)], axis=1)
  elog_ref[...] = elog
  numer_ref[...] = elog * decay


def _sc_denom(elog, lidx3, denom_out, dsh, ebuf, idx2, zbuf):
  c = lax.axis_index("c")
  s = lax.axis_index("s")
  rows_per = NL_PAD // NS

  @pl.when(c == 0)
  def _():
    def zb(i, carry):
      zbuf[i, :] = jnp.zeros((16,), jnp.float32)
      return carry
    lax.fori_loop(0, rows_per, zb, 0)
    pltpu.sync_copy(zbuf, dsh.at[pl.ds(s * rows_per, rows_per)])
    plsc.subcore_barrier()

    pltpu.sync_copy(lidx3.at[pl.ds(2 * s, 2)], idx2)
    row0 = s * (2 * TILE_E)

    def body(j, carry):
      k = j // NCHUNK
      jj = j - k * NCHUNK
      pltpu.sync_copy(elog.at[pl.ds(row0 + j * CHUNK, CHUNK)], ebuf)
      pltpu.sync_copy(ebuf, dsh.at[idx2.at[k, jj]], add=True)
      return carry

    lax.fori_loop(0, 2 * NCHUNK, body, 0)
    plsc.subcore_barrier()
    pltpu.sync_copy(dsh.at[pl.ds(s * rows_per, rows_per)],
                    denom_out.at[pl.ds(s * rows_per, rows_per)])


def _sc_alpha(numer, denom, lidx3, alpha_out, idxv, nbuf, dbuf, abuf, sem):
  w = _widx()
  pltpu.sync_copy(lidx3.at[w], idxv)
  row0 = w * TILE_E

  def body(j, carry):
    r = row0 + j * CHUNK
    pltpu.sync_copy(numer.at[pl.ds(r, CHUNK)], nbuf)
    pltpu.async_copy(denom.at[idxv.at[j]], dbuf, sem).wait()

    def rb(i, c2):
      nv = nbuf[i, :]
      dv = dbuf[i, :]
      abuf[i, :] = nv / (dv + 1e-9)
      return c2

    lax.fori_loop(0, CHUNK, rb, 0)
    pltpu.sync_copy(abuf, alpha_out.at[pl.ds(r, CHUNK)])
    return carry

  lax.fori_loop(0, NCHUNK, body, 0)


def _tc_proj(hp_ref, hl_ref, rp_ref, rl_ref, alpha_ref,
             p1p_ref, p1l_ref, p1g_ref, pb1_ref, p2_ref, pb2_ref,
             msg_ref):
  hp = hp_ref[...]
  hl = hl_ref[...]
  geo, _ = _geometry(rp_ref[...], rl_ref[...])
  msg = jnp.zeros((BE, D), jnp.float32)
  for h in range(H):
    acc = (jnp.dot(hp, p1p_ref[h], preferred_element_type=jnp.float32)
           + jnp.dot(hl, p1l_ref[h], preferred_element_type=jnp.float32)
           + jnp.dot(geo, p1g_ref[h], preferred_element_type=jnp.float32)
           + pb1_ref[h])
    z = acc * jax.nn.sigmoid(acc)
    v = jnp.dot(z, p2_ref[h], preferred_element_type=jnp.float32) + pb2_ref[h]
    msg = msg + v * alpha_ref[:, h:h + 1]
  msg_ref[...] = msg


def _sc_agg(msg, lidx3, agg_out, ash, mbuf, idx2, idxbuf, zbuf):
  c = lax.axis_index("c")
  s = lax.axis_index("s")
  zrows = 64
  per_tile = AGG_PAD // NS

  def zb(i, carry):
    r = i // 16
    o = (i - r * 16) * 16
    zbuf[r, pl.ds(o, 16)] = jnp.zeros((16,), jnp.float32)
    return carry
  lax.fori_loop(0, zrows * (D // 16), zb, 0)
  for t in range(per_tile // zrows):
    pltpu.sync_copy(zbuf, ash.at[pl.ds(s * per_tile + t * zrows, zrows)])
  plsc.subcore_barrier()

  pltpu.sync_copy(lidx3.at[pl.ds(2 * s, 2)], idx2)
  base = c * HALF
  row0 = s * (2 * TILE_E)

  def body(j, carry):
    k = j // NCHUNK
    jj = j - k * NCHUNK
    pltpu.sync_copy(msg.at[pl.ds(row0 + j * CHUNK, CHUNK)], mbuf)
    for g in range(CHUNK // 16):
      v = idx2[k, jj, pl.ds(g * 16, 16)]
      loc = v - base
      ok = (loc >= 0) & (loc < HALF)
      idxbuf[pl.ds(g * 16, 16)] = jnp.where(ok, loc, -1)
    pltpu.sync_copy(mbuf, ash.at[plsc.Indices(idxbuf, ignored_value=-1)],
                    add=True)
    return carry

  lax.fori_loop(0, 2 * NCHUNK, body, 0)
  plsc.subcore_barrier()
  pltpu.sync_copy(ash.at[pl.ds(s * per_tile, per_tile)],
                  agg_out.at[c, pl.ds(s * per_tile, per_tile)])


def _tc_ln(hlig_ref, agg_ref, g_ref, b_ref, out_ref):
  x = hlig_ref[...] + agg_ref[0]
  mean = jnp.mean(x, axis=1, keepdims=True)
  xc = x - mean
  var = jnp.mean(xc * xc, axis=1, keepdims=True)
  out_ref[...] = xc * lax.rsqrt(var + 1e-5) * g_ref[...] + b_ref[...]


def kernel(h_prot, h_lig, cross_edges, prot_pos, lig_pos,
           att_W1, att_b1, att_W2, att_b2,
           proj_W1, proj_b1, proj_W2, proj_b2,
           gamma_l, beta_l):
  f32 = jnp.float32
  p_idx = cross_edges[0].astype(jnp.int32)
  l_idx = cross_edges[1].astype(jnp.int32)
  pidx3 = jnp.pad(p_idx, (0, E_PAD - E)).reshape(NW, NCHUNK, CHUNK)
  lidx3 = jnp.pad(l_idx, (0, E_PAD - E),
                  constant_values=N_L).reshape(NW, NCHUNK, CHUNK)
  tabl = jnp.pad(h_lig, ((0, NL_PAD - N_L), (0, 0)))
  posp = jnp.pad(prot_pos, ((0, 0), (0, PW - 3)))
  posl = jnp.pad(lig_pos, ((0, NL_PAD - N_L), (0, PW - 3)))

  w1p = att_W1[:, :D, :]
  w1l = att_W1[:, D:2 * D, :]
  w1g = jnp.pad(att_W1[:, 2 * D:, :], ((0, 0), (0, GEO - 22), (0, 0)))
  b1 = att_b1.reshape(H, 1, HID)
  w2 = jnp.transpose(att_W2, (0, 2, 1))
  b2 = att_b2.reshape(H, 1, 1)
  p1p = proj_W1[:, :D, :]
  p1l = proj_W1[:, D:2 * D, :]
  p1g = jnp.pad(proj_W1[:, 2 * D:, :], ((0, 0), (0, GEO - 22), (0, 0)))
  pb1 = proj_b1.reshape(H, 1, HID)
  pb2 = proj_b2.reshape(H, 1, D)
  gam = gamma_l.reshape(1, D)
  bet = beta_l.reshape(1, D)

  mesh = plsc.VectorSubcoreMesh(
      core_axis_name="c", subcore_axis_name="s",
      num_cores=NC, num_subcores=NS)

  gather_call = pl.kernel(
      _sc_gather,
      out_type=[
          jax.ShapeDtypeStruct((E_PAD, D), f32),
          jax.ShapeDtypeStruct((E_PAD, D), f32),
          jax.ShapeDtypeStruct((E_PAD, PW), f32),
          jax.ShapeDtypeStruct((E_PAD, PW), f32),
      ],
      mesh=mesh,
      scratch_types=[
          pltpu.VMEM((NCHUNK, CHUNK), jnp.int32),
          pltpu.VMEM((NCHUNK, CHUNK), jnp.int32),
          pltpu.VMEM((CHUNK, D), f32),
          pltpu.VMEM((CHUNK, D), f32),
          pltpu.VMEM((CHUNK, PW), f32),
          pltpu.VMEM((CHUNK, PW), f32),
          pltpu.SemaphoreType.DMA,
          pltpu.SemaphoreType.DMA,
          pltpu.SemaphoreType.DMA,
          pltpu.SemaphoreType.DMA,
      ],
      name="sc_edge_gather",
  )
  hp_e, hl_e, rp_e, rl_e = gather_call(h_prot, tabl, posp, posl, pidx3, lidx3)

  wfull = lambda shape: pl.BlockSpec(shape, lambda i: (0,) * len(shape))
  eblk = lambda wdt: pl.BlockSpec((BE, wdt), lambda i: (i, 0))
  elog, numer = pl.pallas_call(
      _tc_att,
      grid=(GRID_E,),
      in_specs=[
          eblk(D), eblk(D), eblk(PW), eblk(PW),
          wfull((H, D, HID)), wfull((H, D, HID)), wfull((H, GEO, HID)),
          wfull((H, 1, HID)), wfull((H, 1, HID)), wfull((H, 1, 1)),
      ],
      out_specs=[eblk(16), eblk(16)],
      out_shape=[
          jax.ShapeDtypeStruct((E_PAD, 16), f32),
          jax.ShapeDtypeStruct((E_PAD, 16), f32),
      ],
      compiler_params=pltpu.CompilerParams(
          dimension_semantics=("arbitrary",)),
      name="tc_att",
  )(hp_e, hl_e, rp_e, rl_e, w1p, w1l, w1g, b1, w2, b2)

  denom_call = pl.kernel(
      _sc_denom,
      out_type=[jax.ShapeDtypeStruct((NL_PAD, 16), f32)],
      mesh=mesh,
      scratch_types=[
          pltpu.VMEM_SHARED((NL_PAD, 16), f32),
          pltpu.VMEM((CHUNK, 16), f32),
          pltpu.VMEM((2, NCHUNK, CHUNK), jnp.int32),
          pltpu.VMEM((NL_PAD // NS, 16), f32),
      ],
      name="sc_denom",
  )
  (denom,) = denom_call(elog, lidx3)

  alpha_call = pl.kernel(
      _sc_alpha,
      out_type=[jax.ShapeDtypeStruct((E_PAD, 16), f32)],
      mesh=mesh,
      scratch_types=[
          pltpu.VMEM((NCHUNK, CHUNK), jnp.int32),
          pltpu.VMEM((CHUNK, 16), f32),
          pltpu.VMEM((CHUNK, 16), f32),
          pltpu.VMEM((CHUNK, 16), f32),
          pltpu.SemaphoreType.DMA,
      ],
      name="sc_alpha",
  )
  (alpha,) = alpha_call(numer, denom, lidx3)

  (msg,) = pl.pallas_call(
      _tc_proj,
      grid=(GRID_E,),
      in_specs=[
          eblk(D), eblk(D), eblk(PW), eblk(PW), eblk(16),
          wfull((H, D, HID)), wfull((H, D, HID)), wfull((H, GEO, HID)),
          wfull((H, 1, HID)), wfull((H, HID, D)), wfull((H, 1, D)),
      ],
      out_specs=[eblk(D)],
      out_shape=[jax.ShapeDtypeStruct((E_PAD, D), f32)],
      compiler_params=pltpu.CompilerParams(
          dimension_semantics=("arbitrary",)),
      name="tc_proj",
  )(hp_e, hl_e, rp_e, rl_e, alpha, p1p, p1l, p1g, pb1, proj_W2, pb2)

  agg_call = pl.kernel(
      _sc_agg,
      out_type=[jax.ShapeDtypeStruct((NC, AGG_PAD, D), f32)],
      mesh=mesh,
      scratch_types=[
          pltpu.VMEM_SHARED((AGG_PAD, D), f32),
          pltpu.VMEM((CHUNK, D), f32),
          pltpu.VMEM((2, NCHUNK, CHUNK), jnp.int32),
          pltpu.VMEM((CHUNK,), jnp.int32),
          pltpu.VMEM((64, D), f32),
      ],
      name="sc_agg_scatter",
  )
  (agg2,) = agg_call(msg, lidx3)

  BN = 40
  nblk = N_L // BN
  h_l_out = pl.pallas_call(
      _tc_ln,
      grid=(nblk,),
      in_specs=[
          pl.BlockSpec((BN, D), lambda i: (i, 0)),
          pl.BlockSpec((1, BN, D), lambda i: (i // (HALF // BN),
                                              i % (HALF // BN), 0)),
          pl.BlockSpec((1, D), lambda i: (0, 0)),
          pl.BlockSpec((1, D), lambda i: (0, 0)),
      ],
      out_specs=pl.BlockSpec((BN, D), lambda i: (i, 0)),
      out_shape=jax.ShapeDtypeStruct((N_L, D), f32),
      compiler_params=pltpu.CompilerParams(
          dimension_semantics=("arbitrary",)),
      name="tc_layernorm",
  )(h_lig, agg2, gam, bet)

  return (h_prot, h_l_out)


# speedup vs baseline: 3.0087x; 3.0087x over previous
import functools

import jax
import jax.numpy as jnp
from jax import lax
from jax.experimental import pallas as pl
from jax.experimental.pallas import tpu as pltpu
from jax.experimental.pallas import tpu_sc as plsc

N_P = 10000
N_L = 10000
E = 160000
D = 256
HID = 256
H = 4
RBF_DIM = 16
SIGMA = 4.0

NC = 2
NS = 16
NW = NC * NS
CHUNK = 128
TILE_E = 5120
NCHUNK = TILE_E // CHUNK
E_PAD = NW * TILE_E
BE = 512
GRID_E = E_PAD // BE
NL_PAD = 10240
HALF = N_L // 2
AGG_PAD = 5120
GEO = 32
PW = 16
TW = 384
QSPAN = 1024


def _widx():
  return lax.axis_index("c") * NS + lax.axis_index("s")


def _sc_gather(tabp, tabl, pidx3, lidx3,
               ep_out, el_out,
               idxp, idxl, bufp, bufl,
               sem0, sem1):
  w = _widx()
  pltpu.sync_copy(pidx3.at[w], idxp)
  pltpu.sync_copy(lidx3.at[w], idxl)
  row0 = w * TILE_E

  def body(j, carry):
    r = row0 + j * CHUNK
    cp0 = pltpu.async_copy(tabp.at[idxp.at[j]], bufp, sem0)
    cp1 = pltpu.async_copy(tabl.at[idxl.at[j]], bufl, sem1)
    cp0.wait()
    pltpu.sync_copy(bufp, ep_out.at[pl.ds(r, CHUNK)])
    cp1.wait()
    pltpu.sync_copy(bufl, el_out.at[pl.ds(r, CHUNK)])
    return carry

  lax.fori_loop(0, NCHUNK, body, 0)


def _geometry(rp, rl):
  diff = rl - rp
  d2 = jnp.sum(diff * diff, axis=1, keepdims=True)
  dist = jnp.sqrt(d2)
  dirs = diff * (1.0 / (dist + 1e-8))
  centers = lax.broadcasted_iota(jnp.int32, (BE, RBF_DIM), 1).astype(
      jnp.float32) * (8.0 / (RBF_DIM - 1))
  gamma_rbf = 1.0 / (2.0 * (8.0 / RBF_DIM) ** 2)
  rbf = jnp.exp(-gamma_rbf * (dist - centers) ** 2)
  geo = jnp.concatenate(
      [dist, dirs[:, :3], rbf, dirs[:, 0:1], dirs[:, 1:2],
       jnp.zeros((BE, GEO - 22), jnp.float32)], axis=1)
  return geo, d2


def _tc_att(ep_ref, el_ref, l_ref,
            w1p_ref, w1l_ref, w1g_ref, b1_ref, w2_ref, b2_ref,
            elog_ref, den_ref):
  i = pl.program_id(0)

  @pl.when(i == 0)
  def _():
    den_ref[...] = jnp.zeros_like(den_ref)

  hp = ep_ref[:, :D]
  hl = el_ref[:, :D]
  geo, d2 = _geometry(ep_ref[:, D:D + PW], el_ref[:, D:D + PW])
  cols = []
  for h in range(H):
    acc = (jnp.dot(hp, w1p_ref[h], preferred_element_type=jnp.float32)
           + jnp.dot(hl, w1l_ref[h], preferred_element_type=jnp.float32)
           + jnp.dot(geo, w1g_ref[h], preferred_element_type=jnp.float32)
           + b1_ref[h])
    z = acc * jax.nn.sigmoid(acc)
    lg = jnp.sum(z * w2_ref[h], axis=1, keepdims=True) + b2_ref[h]
    cols.append(jnp.exp(lg))
  elog = jnp.concatenate(
      cols + [jnp.zeros((BE, 16 - H), jnp.float32)], axis=1)
  elog_ref[...] = elog
  lcol = l_ref[...]
  for qs in range(0, NL_PAD, QSPAN):
    cols_i = lax.broadcasted_iota(jnp.int32, (BE, QSPAN), 1) + qs
    oh = (lcol == cols_i).astype(jnp.float32)
    upd = lax.dot_general(oh, elog, (((0,), (0,)), ((), ())),
                          preferred_element_type=jnp.float32)
    den_ref[pl.ds(qs, QSPAN), :] += upd


def _sc_den_gather(denT, lidx3, den_out, idxv, dbuf, sem):
  w = _widx()
  pltpu.sync_copy(lidx3.at[w], idxv)
  row0 = w * TILE_E

  def body(j, carry):
    pltpu.async_copy(denT.at[idxv.at[j]], dbuf, sem).wait()
    pltpu.sync_copy(dbuf, den_out.at[pl.ds(row0 + j * CHUNK, CHUNK)])
    return carry

  lax.fori_loop(0, NCHUNK, body, 0)


def _tc_proj(ep_ref, el_ref, l_ref, elog_ref, den_ref,
             p1p_ref, p1l_ref, p1g_ref, pb1_ref, p2_ref, pb2_ref,
             agg_ref):
  i = pl.program_id(0)

  @pl.when(i == 0)
  def _():
    agg_ref[...] = jnp.zeros_like(agg_ref)

  hp = ep_ref[:, :D]
  hl = el_ref[:, :D]
  geo, d2 = _geometry(ep_ref[:, D:D + PW], el_ref[:, D:D + PW])
  decay = jnp.exp(d2 * (-1.0 / (2.0 * SIGMA * SIGMA)))
  alpha = elog_ref[...] * decay / (den_ref[:, :16] + 1e-9)
  msg = jnp.zeros((BE, D), jnp.float32)
  for h in range(H):
    acc = (jnp.dot(hp, p1p_ref[h], preferred_element_type=jnp.float32)
           + jnp.dot(hl, p1l_ref[h], preferred_element_type=jnp.float32)
           + jnp.dot(geo, p1g_ref[h], preferred_element_type=jnp.float32)
           + pb1_ref[h])
    z = acc * jax.nn.sigmoid(acc)
    v = jnp.dot(z, p2_ref[h], preferred_element_type=jnp.float32) + pb2_ref[h]
    msg = msg + v * alpha[:, h:h + 1]
  lcol = l_ref[...]
  for qs in range(0, NL_PAD, QSPAN):
    cols_i = lax.broadcasted_iota(jnp.int32, (BE, QSPAN), 1) + qs
    oh = (lcol == cols_i).astype(jnp.float32)
    upd = lax.dot_general(oh, msg, (((0,), (0,)), ((), ())),
                          preferred_element_type=jnp.float32)
    agg_ref[pl.ds(qs, QSPAN), :] += upd


def _tc_ln(hlig_ref, agg_ref, g_ref, b_ref, out_ref):
  x = hlig_ref[...] + agg_ref[...]
  mean = jnp.mean(x, axis=1, keepdims=True)
  xc = x - mean
  var = jnp.mean(xc * xc, axis=1, keepdims=True)
  out_ref[...] = xc * lax.rsqrt(var + 1e-5) * g_ref[...] + b_ref[...]


def kernel(h_prot, h_lig, cross_edges, prot_pos, lig_pos,
           att_W1, att_b1, att_W2, att_b2,
           proj_W1, proj_b1, proj_W2, proj_b2,
           gamma_l, beta_l):
  f32 = jnp.float32
  p_idx = cross_edges[0].astype(jnp.int32)
  l_idx = cross_edges[1].astype(jnp.int32)
  pidx3 = jnp.pad(p_idx, (0, E_PAD - E)).reshape(NW, NCHUNK, CHUNK)
  lpad = jnp.pad(l_idx, (0, E_PAD - E), constant_values=N_L)
  lidx3 = lpad.reshape(NW, NCHUNK, CHUNK)
  tabp = jnp.concatenate(
      [h_prot, prot_pos, jnp.zeros((N_P, TW - D - 3), f32)], axis=1)
  tabl = jnp.pad(
      jnp.concatenate(
          [h_lig, lig_pos, jnp.zeros((N_L, TW - D - 3), f32)], axis=1),
      ((0, NL_PAD - N_L), (0, 0)))

  w1p = att_W1[:, :D, :]
  w1l = att_W1[:, D:2 * D, :]
  w1g = jnp.pad(att_W1[:, 2 * D:, :], ((0, 0), (0, GEO - 22), (0, 0)))
  b1 = att_b1.reshape(H, 1, HID)
  w2 = jnp.transpose(att_W2, (0, 2, 1))
  b2 = att_b2.reshape(H, 1, 1)
  p1p = proj_W1[:, :D, :]
  p1l = proj_W1[:, D:2 * D, :]
  p1g = jnp.pad(proj_W1[:, 2 * D:, :], ((0, 0), (0, GEO - 22), (0, 0)))
  pb1 = proj_b1.reshape(H, 1, HID)
  pb2 = proj_b2.reshape(H, 1, D)
  gam = gamma_l.reshape(1, D)
  bet = beta_l.reshape(1, D)

  mesh = plsc.VectorSubcoreMesh(
      core_axis_name="c", subcore_axis_name="s",
      num_cores=NC, num_subcores=NS)

  gather_call = pl.kernel(
      _sc_gather,
      out_type=[
          jax.ShapeDtypeStruct((E_PAD, TW), f32),
          jax.ShapeDtypeStruct((E_PAD, TW), f32),
      ],
      mesh=mesh,
      scratch_types=[
          pltpu.VMEM((NCHUNK, CHUNK), jnp.int32),
          pltpu.VMEM((NCHUNK, CHUNK), jnp.int32),
          pltpu.VMEM((CHUNK, TW), f32),
          pltpu.VMEM((CHUNK, TW), f32),
          pltpu.SemaphoreType.DMA,
          pltpu.SemaphoreType.DMA,
      ],
      name="sc_edge_gather",
  )
  ep_e, el_e = gather_call(tabp, tabl, pidx3, lidx3)

  wfull = lambda shape: pl.BlockSpec(shape, lambda i: (0,) * len(shape))
  eblk = lambda wdt: pl.BlockSpec((BE, wdt), lambda i: (i, 0))
  lcol2 = lpad.reshape(E_PAD, 1)
  elog, den = pl.pallas_call(
      _tc_att,
      grid=(GRID_E,),
      in_specs=[
          eblk(TW), eblk(TW), eblk(1),
          wfull((H, D, HID)), wfull((H, D, HID)), wfull((H, GEO, HID)),
          wfull((H, 1, HID)), wfull((H, 1, HID)), wfull((H, 1, 1)),
      ],
      out_specs=[eblk(16), pl.BlockSpec((NL_PAD, 16), lambda i: (0, 0))],
      out_shape=[
          jax.ShapeDtypeStruct((E_PAD, 16), f32),
          jax.ShapeDtypeStruct((NL_PAD, 16), f32),
      ],
      compiler_params=pltpu.CompilerParams(
          dimension_semantics=("arbitrary",)),
      name="tc_att",
  )(ep_e, el_e, lcol2, w1p, w1l, w1g, b1, w2, b2)
  denT = jnp.pad(den, ((0, 0), (0, 128 - 16)))

  dengather_call = pl.kernel(
      _sc_den_gather,
      out_type=[jax.ShapeDtypeStruct((E_PAD, 128), f32)],
      mesh=mesh,
      scratch_types=[
          pltpu.VMEM((NCHUNK, CHUNK), jnp.int32),
          pltpu.VMEM((CHUNK, 128), f32),
          pltpu.SemaphoreType.DMA,
      ],
      name="sc_den_gather",
  )
  (den_e,) = dengather_call(denT, lidx3)

  (agg,) = pl.pallas_call(
      _tc_proj,
      grid=(GRID_E,),
      in_specs=[
          eblk(TW), eblk(TW), eblk(1), eblk(16), eblk(128),
          wfull((H, D, HID)), wfull((H, D, HID)), wfull((H, GEO, HID)),
          wfull((H, 1, HID)), wfull((H, HID, D)), wfull((H, 1, D)),
      ],
      out_specs=[pl.BlockSpec((NL_PAD, D), lambda i: (0, 0))],
      out_shape=[jax.ShapeDtypeStruct((NL_PAD, D), f32)],
      compiler_params=pltpu.CompilerParams(
          dimension_semantics=("arbitrary",)),
      name="tc_proj",
  )(ep_e, el_e, lcol2, elog, den_e, p1p, p1l, p1g, pb1, proj_W2, pb2)

  BN = 40
  nblk = N_L // BN
  h_l_out = pl.pallas_call(
      _tc_ln,
      grid=(nblk,),
      in_specs=[
          pl.BlockSpec((BN, D), lambda i: (i, 0)),
          pl.BlockSpec((BN, D), lambda i: (i, 0)),
          pl.BlockSpec((1, D), lambda i: (0, 0)),
          pl.BlockSpec((1, D), lambda i: (0, 0)),
      ],
      out_specs=pl.BlockSpec((BN, D), lambda i: (i, 0)),
      out_shape=jax.ShapeDtypeStruct((N_L, D), f32),
      compiler_params=pltpu.CompilerParams(
          dimension_semantics=("arbitrary",)),
      name="tc_layernorm",
  )(h_lig, agg, gam, bet)

  return (h_prot, h_l_out)
